# Initial kernel scaffold; baseline (speedup 1.0000x reference)
import jax
import jax.numpy as jnp
from jax.experimental import pallas as pl


def kernel(feature, adj, gcn_W1, gcn_b1, gcn_W2, gcn_b2, gat_W, gat_a, out_W, out_a):
    raise NotImplementedError("write your pallas kernel here")


# speedup vs baseline: 1.3979x; 1.3979x over previous
import functools

import jax
import jax.numpy as jnp
from jax import lax
from jax.experimental import pallas as pl

ALPHA = 0.1
NEG = -9e15


def _mm_kernel(x_ref, w_ref, out_ref):
    out_ref[...] = jnp.dot(x_ref[...], w_ref[...],
                           preferred_element_type=jnp.float32)


def _small_matmul(x, w):
    m, _ = x.shape
    n = w.shape[1]
    return pl.pallas_call(
        _mm_kernel,
        out_shape=jax.ShapeDtypeStruct((m, n), jnp.float32),
    )(x, w)


def _gcn_body(adj_ref, p_ref, b_ref, out_ref, *, relu):
    a = adj_ref[...].astype(jnp.float32)
    x = jnp.dot(a, p_ref[...], preferred_element_type=jnp.float32)
    x = x + b_ref[...]
    if relu:
        x = jnp.maximum(x, 0.0)
    out_ref[...] = x


def _gcn_layer(adj_b, p, b, relu, br):
    n = adj_b.shape[0]
    h = p.shape[1]
    return pl.pallas_call(
        functools.partial(_gcn_body, relu=relu),
        grid=(n // br,),
        in_specs=[
            pl.BlockSpec((br, n), lambda i: (i, 0)),
            pl.BlockSpec((n, h), lambda i: (0, 0)),
            pl.BlockSpec((1, h), lambda i: (0, 0)),
        ],
        out_specs=pl.BlockSpec((br, h), lambda i: (i, 0)),
        out_shape=jax.ShapeDtypeStruct((n, h), jnp.float32),
    )(adj_b, p, b)


def _head_pre_body(x_ref, w_ref, a1_ref, a2_ref, wh_ref, f1_ref, f2t_ref):
    wh = jnp.dot(x_ref[...], w_ref[0], preferred_element_type=jnp.float32)
    wh_ref[0] = wh
    f1_ref[0] = lax.dot_general(wh, a1_ref[0], (((1,), (1,)), ((), ())),
                                preferred_element_type=jnp.float32)
    f2t_ref[0] = lax.dot_general(a2_ref[0], wh, (((1,), (1,)), ((), ())),
                                 preferred_element_type=jnp.float32)


def _head_pre(x, gat_w, a1, a2):
    heads, feat, hid = gat_w.shape
    n = x.shape[0]
    return pl.pallas_call(
        _head_pre_body,
        grid=(heads,),
        in_specs=[
            pl.BlockSpec((n, feat), lambda h: (0, 0)),
            pl.BlockSpec((1, feat, hid), lambda h: (h, 0, 0)),
            pl.BlockSpec((1, 1, hid), lambda h: (h, 0, 0)),
            pl.BlockSpec((1, 1, hid), lambda h: (h, 0, 0)),
        ],
        out_specs=[
            pl.BlockSpec((1, n, hid), lambda h: (h, 0, 0)),
            pl.BlockSpec((1, n, 1), lambda h: (h, 0, 0)),
            pl.BlockSpec((1, 1, n), lambda h: (h, 0, 0)),
        ],
        out_shape=[
            jax.ShapeDtypeStruct((heads, n, hid), jnp.float32),
            jax.ShapeDtypeStruct((heads, n, 1), jnp.float32),
            jax.ShapeDtypeStruct((heads, 1, n), jnp.float32),
        ],
    )(x, gat_w, a1, a2)


def _row_softmax_masked(e, mask):
    e = jnp.where(e > 0, e, ALPHA * e)
    e = jnp.where(mask, e, NEG)
    m = jnp.max(e, axis=1, keepdims=True)
    ex = jnp.exp(e - m)
    s = jnp.sum(ex, axis=1, keepdims=True)
    return ex / s


def _gat_heads_body(adj_ref, wh_ref, f1_ref, f2t_ref, out_ref, *, heads, hid):
    mask = adj_ref[...] > 0
    for h in range(heads):
        att = _row_softmax_masked(f1_ref[h] + f2t_ref[h], mask)
        hp = jnp.dot(att, wh_ref[h], preferred_element_type=jnp.float32)
        out_ref[:, h * hid:(h + 1) * hid] = jnp.where(hp > 0, hp,
                                                      jnp.exp(hp) - 1.0)


def _gat_heads(adj_b, wh_all, f1_all, f2t_all, br):
    heads, n, hid = wh_all.shape
    return pl.pallas_call(
        functools.partial(_gat_heads_body, heads=heads, hid=hid),
        grid=(n // br,),
        in_specs=[
            pl.BlockSpec((br, n), lambda i: (i, 0)),
            pl.BlockSpec((heads, n, hid), lambda i: (0, 0, 0)),
            pl.BlockSpec((heads, br, 1), lambda i: (0, i, 0)),
            pl.BlockSpec((heads, 1, n), lambda i: (0, 0, 0)),
        ],
        out_specs=pl.BlockSpec((br, heads * hid), lambda i: (i, 0)),
        out_shape=jax.ShapeDtypeStruct((n, heads * hid), jnp.float32),
    )(adj_b, wh_all, f1_all, f2t_all)


def _out_pre_body(x_ref, w_ref, a1_ref, a2_ref, wh_ref, f1_ref, f2t_ref):
    wh = jnp.dot(x_ref[...], w_ref[...], preferred_element_type=jnp.float32)
    wh_ref[...] = wh
    f1_ref[...] = lax.dot_general(wh, a1_ref[...], (((1,), (1,)), ((), ())),
                                  preferred_element_type=jnp.float32)
    f2t_ref[...] = lax.dot_general(a2_ref[...], wh, (((1,), (1,)), ((), ())),
                                   preferred_element_type=jnp.float32)


def _out_pre(x, out_w, oa1, oa2):
    n = x.shape[0]
    feat = out_w.shape[1]
    return pl.pallas_call(
        _out_pre_body,
        out_shape=[
            jax.ShapeDtypeStruct((n, feat), jnp.float32),
            jax.ShapeDtypeStruct((n, 1), jnp.float32),
            jax.ShapeDtypeStruct((1, n), jnp.float32),
        ],
    )(x, out_w, oa1, oa2)


def _gat_out_body(adj_ref, wh_ref, f1_ref, f2t_ref, out_ref):
    mask = adj_ref[...] > 0
    att = _row_softmax_masked(f1_ref[...] + f2t_ref[...], mask)
    hp = jnp.dot(att, wh_ref[...], preferred_element_type=jnp.float32)
    y = jnp.where(hp > 0, hp, jnp.exp(hp) - 1.0)
    my = jnp.max(y, axis=1, keepdims=True)
    lse = jnp.log(jnp.sum(jnp.exp(y - my), axis=1, keepdims=True)) + my
    out_ref[...] = y - lse


def _gat_out(adj_b, wh, f1, f2t, br):
    n, feat = wh.shape
    return pl.pallas_call(
        _gat_out_body,
        grid=(n // br,),
        in_specs=[
            pl.BlockSpec((br, n), lambda i: (i, 0)),
            pl.BlockSpec((n, feat), lambda i: (0, 0)),
            pl.BlockSpec((br, 1), lambda i: (i, 0)),
            pl.BlockSpec((1, n), lambda i: (0, 0)),
        ],
        out_specs=pl.BlockSpec((br, feat), lambda i: (i, 0)),
        out_shape=jax.ShapeDtypeStruct((n, feat), jnp.float32),
    )(adj_b, wh, f1, f2t)


def kernel(feature, adj, gcn_W1, gcn_b1, gcn_W2, gcn_b2, gat_W, gat_a,
           out_W, out_a):
    n, feat = feature.shape
    hid = gcn_W1.shape[1]
    heads = gat_W.shape[0]
    br = 256 if n % 256 == 0 else n

    adj_b = adj.astype(jnp.bfloat16)
    b1 = gcn_b1.reshape(1, hid)
    b2 = gcn_b2.reshape(1, feat)

    p1 = _small_matmul(feature, gcn_W1)
    x1 = _gcn_layer(adj_b, p1, b1, True, br)
    p2 = _small_matmul(x1, gcn_W2)
    x2 = _gcn_layer(adj_b, p2, b2, False, br)

    a1 = gat_a[:, :hid, 0].reshape(heads, 1, hid)
    a2 = gat_a[:, hid:, 0].reshape(heads, 1, hid)
    wh_all, f1_all, f2t_all = _head_pre(x2, gat_W, a1, a2)
    x_cat = _gat_heads(adj_b, wh_all, f1_all, f2t_all, br)

    oa1 = out_a[:feat, 0].reshape(1, feat)
    oa2 = out_a[feat:, 0].reshape(1, feat)
    wh_o, f1_o, f2t_o = _out_pre(x_cat, out_W, oa1, oa2)
    x_out = _gat_out(adj_b, wh_o, f1_o, f2t_o, br)

    return (x_out, adj)


# speedup vs baseline: 1.4205x; 1.0161x over previous
import functools

import jax
import jax.numpy as jnp
from jax import lax
from jax.experimental import pallas as pl

ALPHA = 0.1
NEG = -9e15


def _mm_kernel(x_ref, w_ref, out_ref):
    out_ref[...] = jnp.dot(x_ref[...], w_ref[...],
                           preferred_element_type=jnp.float32)


def _small_matmul(x, w):
    m, _ = x.shape
    n = w.shape[1]
    return pl.pallas_call(
        _mm_kernel,
        out_shape=jax.ShapeDtypeStruct((m, n), jnp.float32),
    )(x, w)


def _gcn_body(adj_ref, p_ref, b_ref, out_ref, *, relu):
    a = adj_ref[...].astype(jnp.float32)
    x = jnp.dot(a, p_ref[...], preferred_element_type=jnp.float32)
    x = x + b_ref[...]
    if relu:
        x = jnp.maximum(x, 0.0)
    out_ref[...] = x


def _gcn_layer(adj_b, p, b, relu, br):
    n = adj_b.shape[0]
    h = p.shape[1]
    return pl.pallas_call(
        functools.partial(_gcn_body, relu=relu),
        grid=(n // br,),
        in_specs=[
            pl.BlockSpec((br, n), lambda i: (i, 0)),
            pl.BlockSpec((n, h), lambda i: (0, 0)),
            pl.BlockSpec((1, h), lambda i: (0, 0)),
        ],
        out_specs=pl.BlockSpec((br, h), lambda i: (i, 0)),
        out_shape=jax.ShapeDtypeStruct((n, h), jnp.float32),
    )(adj_b, p, b)


def _head_pre_body(x_ref, w_ref, a1_ref, a2_ref, wh_ref, f1_ref, f2t_ref):
    wh = jnp.dot(x_ref[...], w_ref[0], preferred_element_type=jnp.float32)
    wh_ref[0] = wh
    f1_ref[0] = lax.dot_general(wh, a1_ref[0], (((1,), (1,)), ((), ())),
                                preferred_element_type=jnp.float32)
    f2t_ref[0] = lax.dot_general(a2_ref[0], wh, (((1,), (1,)), ((), ())),
                                 preferred_element_type=jnp.float32)


def _head_pre(x, gat_w, a1, a2):
    heads, feat, hid = gat_w.shape
    n = x.shape[0]
    return pl.pallas_call(
        _head_pre_body,
        grid=(heads,),
        in_specs=[
            pl.BlockSpec((n, feat), lambda h: (0, 0)),
            pl.BlockSpec((1, feat, hid), lambda h: (h, 0, 0)),
            pl.BlockSpec((1, 1, hid), lambda h: (h, 0, 0)),
            pl.BlockSpec((1, 1, hid), lambda h: (h, 0, 0)),
        ],
        out_specs=[
            pl.BlockSpec((1, n, hid), lambda h: (h, 0, 0)),
            pl.BlockSpec((1, n, 1), lambda h: (h, 0, 0)),
            pl.BlockSpec((1, 1, n), lambda h: (h, 0, 0)),
        ],
        out_shape=[
            jax.ShapeDtypeStruct((heads, n, hid), jnp.float32),
            jax.ShapeDtypeStruct((heads, n, 1), jnp.float32),
            jax.ShapeDtypeStruct((heads, 1, n), jnp.float32),
        ],
    )(x, gat_w, a1, a2)


def _row_softmax_unnorm(e, mask):
    e = jnp.maximum(e, ALPHA * e)
    e = jnp.where(mask, e, NEG)
    m = jnp.max(e, axis=1, keepdims=True)
    ex = jnp.exp(e - m)
    s = jnp.sum(ex, axis=1, keepdims=True)
    return ex, s


def _gat_heads_body(adj_ref, wh_ref, f1_ref, f2t_ref, out_ref, *, heads, hid):
    mask = adj_ref[...] > 0
    for h in range(heads):
        p, s = _row_softmax_unnorm(f1_ref[h] + f2t_ref[h], mask)
        hp = jnp.dot(p, wh_ref[h], preferred_element_type=jnp.float32) / s
        out_ref[:, h * hid:(h + 1) * hid] = jnp.where(hp > 0, hp,
                                                      jnp.exp(hp) - 1.0)


def _gat_heads(adj_b, wh_all, f1_all, f2t_all, br):
    heads, n, hid = wh_all.shape
    return pl.pallas_call(
        functools.partial(_gat_heads_body, heads=heads, hid=hid),
        grid=(n // br,),
        in_specs=[
            pl.BlockSpec((br, n), lambda i: (i, 0)),
            pl.BlockSpec((heads, n, hid), lambda i: (0, 0, 0)),
            pl.BlockSpec((heads, br, 1), lambda i: (0, i, 0)),
            pl.BlockSpec((heads, 1, n), lambda i: (0, 0, 0)),
        ],
        out_specs=pl.BlockSpec((br, heads * hid), lambda i: (i, 0)),
        out_shape=jax.ShapeDtypeStruct((n, heads * hid), jnp.float32),
    )(adj_b, wh_all, f1_all, f2t_all)


def _out_pre_body(x_ref, w_ref, a1_ref, a2_ref, wh_ref, f1_ref, f2t_ref):
    wh = jnp.dot(x_ref[...], w_ref[...], preferred_element_type=jnp.float32)
    wh_ref[...] = wh
    f1_ref[...] = lax.dot_general(wh, a1_ref[...], (((1,), (1,)), ((), ())),
                                  preferred_element_type=jnp.float32)
    f2t_ref[...] = lax.dot_general(a2_ref[...], wh, (((1,), (1,)), ((), ())),
                                   preferred_element_type=jnp.float32)


def _out_pre(x, out_w, oa1, oa2):
    n = x.shape[0]
    feat = out_w.shape[1]
    return pl.pallas_call(
        _out_pre_body,
        out_shape=[
            jax.ShapeDtypeStruct((n, feat), jnp.float32),
            jax.ShapeDtypeStruct((n, 1), jnp.float32),
            jax.ShapeDtypeStruct((1, n), jnp.float32),
        ],
    )(x, out_w, oa1, oa2)


def _gat_out_body(adj_ref, wh_ref, f1_ref, f2t_ref, out_ref):
    mask = adj_ref[...] > 0
    p, s = _row_softmax_unnorm(f1_ref[...] + f2t_ref[...], mask)
    hp = jnp.dot(p, wh_ref[...], preferred_element_type=jnp.float32) / s
    y = jnp.where(hp > 0, hp, jnp.exp(hp) - 1.0)
    my = jnp.max(y, axis=1, keepdims=True)
    lse = jnp.log(jnp.sum(jnp.exp(y - my), axis=1, keepdims=True)) + my
    out_ref[...] = y - lse


def _gat_out(adj_b, wh, f1, f2t, br):
    n, feat = wh.shape
    return pl.pallas_call(
        _gat_out_body,
        grid=(n // br,),
        in_specs=[
            pl.BlockSpec((br, n), lambda i: (i, 0)),
            pl.BlockSpec((n, feat), lambda i: (0, 0)),
            pl.BlockSpec((br, 1), lambda i: (i, 0)),
            pl.BlockSpec((1, n), lambda i: (0, 0)),
        ],
        out_specs=pl.BlockSpec((br, feat), lambda i: (i, 0)),
        out_shape=jax.ShapeDtypeStruct((n, feat), jnp.float32),
    )(adj_b, wh, f1, f2t)


def kernel(feature, adj, gcn_W1, gcn_b1, gcn_W2, gcn_b2, gat_W, gat_a,
           out_W, out_a):
    n, feat = feature.shape
    hid = gcn_W1.shape[1]
    heads = gat_W.shape[0]
    br = 256 if n % 256 == 0 else n

    adj_b = adj.astype(jnp.bfloat16)
    b1 = gcn_b1.reshape(1, hid)
    b2 = gcn_b2.reshape(1, feat)

    p1 = _small_matmul(feature, gcn_W1)
    x1 = _gcn_layer(adj_b, p1, b1, True, br)
    p2 = _small_matmul(x1, gcn_W2)
    x2 = _gcn_layer(adj_b, p2, b2, False, br)

    a1 = gat_a[:, :hid, 0].reshape(heads, 1, hid)
    a2 = gat_a[:, hid:, 0].reshape(heads, 1, hid)
    wh_all, f1_all, f2t_all = _head_pre(x2, gat_W, a1, a2)
    x_cat = _gat_heads(adj_b, wh_all, f1_all, f2t_all, br)

    oa1 = out_a[:feat, 0].reshape(1, feat)
    oa2 = out_a[feat:, 0].reshape(1, feat)
    wh_o, f1_o, f2t_o = _out_pre(x_cat, out_W, oa1, oa2)
    x_out = _gat_out(adj_b, wh_o, f1_o, f2t_o, br)

    return (x_out, adj)


# speedup vs baseline: 1.4923x; 1.0506x over previous
import functools

import jax
import jax.numpy as jnp
from jax import lax
from jax.experimental import pallas as pl
from jax.experimental.pallas import tpu as pltpu

ALPHA = 0.1
NEG = -9e15


def _row_softmax_unnorm(e, mask):
    e = jnp.maximum(e, ALPHA * e)
    e = jnp.where(mask, e, NEG)
    m = jnp.max(e, axis=1, keepdims=True)
    ex = jnp.exp(e - m)
    s = jnp.sum(ex, axis=1, keepdims=True)
    return ex, s


def _k1_body(feat_ref, w1_ref, b1_ref, adj_ref, x1_ref, adj16_ref, p1_scr):
    @pl.when(pl.program_id(0) == 0)
    def _():
        p1_scr[...] = jnp.dot(feat_ref[...], w1_ref[...],
                              preferred_element_type=jnp.float32)

    a = adj_ref[...]
    adj16_ref[...] = a.astype(jnp.bfloat16)
    x = jnp.dot(a, p1_scr[...], preferred_element_type=jnp.float32)
    x1_ref[...] = jnp.maximum(x + b1_ref[...], 0.0)


def _k1(feature, w1, b1, adj, br):
    n, feat = feature.shape
    hid = w1.shape[1]
    return pl.pallas_call(
        _k1_body,
        grid=(n // br,),
        in_specs=[
            pl.BlockSpec((n, feat), lambda i: (0, 0)),
            pl.BlockSpec((feat, hid), lambda i: (0, 0)),
            pl.BlockSpec((1, hid), lambda i: (0, 0)),
            pl.BlockSpec((br, n), lambda i: (i, 0)),
        ],
        out_specs=[
            pl.BlockSpec((br, hid), lambda i: (i, 0)),
            pl.BlockSpec((br, n), lambda i: (i, 0)),
        ],
        out_shape=[
            jax.ShapeDtypeStruct((n, hid), jnp.float32),
            jax.ShapeDtypeStruct((n, n), jnp.bfloat16),
        ],
        scratch_shapes=[pltpu.VMEM((n, hid), jnp.float32)],
    )(feature, w1, b1, adj)


def _k2_body(x1_ref, w2_ref, b2_ref, gatw_ref, a1_ref, a2_ref, adj16_ref,
             wh_ref, f1_ref, f2t_ref, p2_scr, *, heads):
    @pl.when(pl.program_id(0) == 0)
    def _():
        p2_scr[...] = jnp.dot(x1_ref[...], w2_ref[...],
                              preferred_element_type=jnp.float32)

    a = adj16_ref[...].astype(jnp.float32)
    x2 = jnp.dot(a, p2_scr[...], preferred_element_type=jnp.float32)
    x2 = x2 + b2_ref[...]
    for h in range(heads):
        wh = jnp.dot(x2, gatw_ref[h], preferred_element_type=jnp.float32)
        wh_ref[h] = wh
        f1_ref[h] = lax.dot_general(wh, a1_ref[h], (((1,), (1,)), ((), ())),
                                    preferred_element_type=jnp.float32)
        f2t_ref[h] = lax.dot_general(a2_ref[h], wh, (((1,), (1,)), ((), ())),
                                     preferred_element_type=jnp.float32)


def _k2(x1, w2, b2, gat_w, a1, a2, adj16, br):
    n, hid = x1.shape
    heads, feat, ghid = gat_w.shape
    return pl.pallas_call(
        functools.partial(_k2_body, heads=heads),
        grid=(n // br,),
        in_specs=[
            pl.BlockSpec((n, hid), lambda i: (0, 0)),
            pl.BlockSpec((hid, feat), lambda i: (0, 0)),
            pl.BlockSpec((1, feat), lambda i: (0, 0)),
            pl.BlockSpec((heads, feat, ghid), lambda i: (0, 0, 0)),
            pl.BlockSpec((heads, 1, ghid), lambda i: (0, 0, 0)),
            pl.BlockSpec((heads, 1, ghid), lambda i: (0, 0, 0)),
            pl.BlockSpec((br, n), lambda i: (i, 0)),
        ],
        out_specs=[
            pl.BlockSpec((heads, br, ghid), lambda i: (0, i, 0)),
            pl.BlockSpec((heads, br, 1), lambda i: (0, i, 0)),
            pl.BlockSpec((heads, 1, br), lambda i: (0, 0, i)),
        ],
        out_shape=[
            jax.ShapeDtypeStruct((heads, n, ghid), jnp.float32),
            jax.ShapeDtypeStruct((heads, n, 1), jnp.float32),
            jax.ShapeDtypeStruct((heads, 1, n), jnp.float32),
        ],
        scratch_shapes=[pltpu.VMEM((n, feat), jnp.float32)],
    )(x1, w2, b2, gat_w, a1, a2, adj16)


def _k3_body(adj16_ref, wh_ref, f1_ref, f2t_ref, outw_ref, oa1_ref, oa2_ref,
             who_ref, f1o_ref, f2to_ref, *, heads):
    mask = adj16_ref[...] > 0
    cats = []
    for h in range(heads):
        p, s = _row_softmax_unnorm(f1_ref[h] + f2t_ref[h], mask)
        hp = jnp.dot(p, wh_ref[h], preferred_element_type=jnp.float32) / s
        cats.append(jnp.where(hp > 0, hp, jnp.exp(hp) - 1.0))
    xcat = jnp.concatenate(cats, axis=1)
    who = jnp.dot(xcat, outw_ref[...], preferred_element_type=jnp.float32)
    who_ref[...] = who
    f1o_ref[...] = lax.dot_general(who, oa1_ref[...], (((1,), (1,)), ((), ())),
                                   preferred_element_type=jnp.float32)
    f2to_ref[...] = lax.dot_general(oa2_ref[...], who, (((1,), (1,)), ((), ())),
                                    preferred_element_type=jnp.float32)


def _k3(adj16, wh_all, f1_all, f2t_all, out_w, oa1, oa2, br):
    heads, n, hid = wh_all.shape
    feat = out_w.shape[1]
    return pl.pallas_call(
        functools.partial(_k3_body, heads=heads),
        grid=(n // br,),
        in_specs=[
            pl.BlockSpec((br, n), lambda i: (i, 0)),
            pl.BlockSpec((heads, n, hid), lambda i: (0, 0, 0)),
            pl.BlockSpec((heads, br, 1), lambda i: (0, i, 0)),
            pl.BlockSpec((heads, 1, n), lambda i: (0, 0, 0)),
            pl.BlockSpec((heads * hid, feat), lambda i: (0, 0)),
            pl.BlockSpec((1, feat), lambda i: (0, 0)),
            pl.BlockSpec((1, feat), lambda i: (0, 0)),
        ],
        out_specs=[
            pl.BlockSpec((br, feat), lambda i: (i, 0)),
            pl.BlockSpec((br, 1), lambda i: (i, 0)),
            pl.BlockSpec((1, br), lambda i: (0, i)),
        ],
        out_shape=[
            jax.ShapeDtypeStruct((n, feat), jnp.float32),
            jax.ShapeDtypeStruct((n, 1), jnp.float32),
            jax.ShapeDtypeStruct((1, n), jnp.float32),
        ],
    )(adj16, wh_all, f1_all, f2t_all, out_w, oa1, oa2)


def _k4_body(adj16_ref, who_ref, f1o_ref, f2to_ref, out_ref):
    mask = adj16_ref[...] > 0
    p, s = _row_softmax_unnorm(f1o_ref[...] + f2to_ref[...], mask)
    hp = jnp.dot(p, who_ref[...], preferred_element_type=jnp.float32) / s
    y = jnp.where(hp > 0, hp, jnp.exp(hp) - 1.0)
    my = jnp.max(y, axis=1, keepdims=True)
    lse = jnp.log(jnp.sum(jnp.exp(y - my), axis=1, keepdims=True)) + my
    out_ref[...] = y - lse


def _k4(adj16, who, f1o, f2to, br):
    n, feat = who.shape
    return pl.pallas_call(
        _k4_body,
        grid=(n // br,),
        in_specs=[
            pl.BlockSpec((br, n), lambda i: (i, 0)),
            pl.BlockSpec((n, feat), lambda i: (0, 0)),
            pl.BlockSpec((br, 1), lambda i: (i, 0)),
            pl.BlockSpec((1, n), lambda i: (0, 0)),
        ],
        out_specs=pl.BlockSpec((br, feat), lambda i: (i, 0)),
        out_shape=jax.ShapeDtypeStruct((n, feat), jnp.float32),
    )(adj16, who, f1o, f2to)


def kernel(feature, adj, gcn_W1, gcn_b1, gcn_W2, gcn_b2, gat_W, gat_a,
           out_W, out_a):
    n, feat = feature.shape
    hid = gcn_W1.shape[1]
    heads = gat_W.shape[0]
    br = 256 if n % 256 == 0 else n

    b1 = gcn_b1.reshape(1, hid)
    b2 = gcn_b2.reshape(1, feat)
    a1 = gat_a[:, :hid, 0].reshape(heads, 1, hid)
    a2 = gat_a[:, hid:, 0].reshape(heads, 1, hid)
    oa1 = out_a[:feat, 0].reshape(1, feat)
    oa2 = out_a[feat:, 0].reshape(1, feat)

    x1, adj16 = _k1(feature, gcn_W1, b1, adj, br)
    wh_all, f1_all, f2t_all = _k2(x1, gcn_W2, b2, gat_W, a1, a2, adj16, br)
    who, f1o, f2to = _k3(adj16, wh_all, f1_all, f2t_all, out_W, oa1, oa2, br)
    x_out = _k4(adj16, who, f1o, f2to, br)

    return (x_out, adj)


# speedup vs baseline: 1.5168x; 1.0164x over previous
import functools

import jax
import jax.numpy as jnp
from jax import lax
from jax.experimental import pallas as pl
from jax.experimental.pallas import tpu as pltpu

ALPHA = 0.1
NEG = -9e15


def _row_softmax_unnorm(e, mask):
    e = jnp.maximum(e, ALPHA * e)
    e = jnp.where(mask, e, NEG)
    m = jnp.max(e, axis=1, keepdims=True)
    ex = jnp.exp(e - m)
    s = jnp.sum(ex, axis=1, keepdims=True)
    return ex * (1.0 / s)


def _k1_body(feat_ref, w1_ref, b1_ref, adj_ref, x1_ref, adj16_ref, p1_scr):
    @pl.when(pl.program_id(0) == 0)
    def _():
        p1_scr[...] = jnp.dot(feat_ref[...], w1_ref[...],
                              preferred_element_type=jnp.float32)

    a = adj_ref[...]
    adj16_ref[...] = a.astype(jnp.bfloat16)
    x = jnp.dot(a, p1_scr[...], preferred_element_type=jnp.float32)
    x1_ref[...] = jnp.maximum(x + b1_ref[...], 0.0)


def _k1(feature, w1, b1, adj, br):
    n, feat = feature.shape
    hid = w1.shape[1]
    return pl.pallas_call(
        _k1_body,
        grid=(n // br,),
        in_specs=[
            pl.BlockSpec((n, feat), lambda i: (0, 0)),
            pl.BlockSpec((feat, hid), lambda i: (0, 0)),
            pl.BlockSpec((1, hid), lambda i: (0, 0)),
            pl.BlockSpec((br, n), lambda i: (i, 0)),
        ],
        out_specs=[
            pl.BlockSpec((br, hid), lambda i: (i, 0)),
            pl.BlockSpec((br, n), lambda i: (i, 0)),
        ],
        out_shape=[
            jax.ShapeDtypeStruct((n, hid), jnp.float32),
            jax.ShapeDtypeStruct((n, n), jnp.bfloat16),
        ],
        scratch_shapes=[pltpu.VMEM((n, hid), jnp.float32)],
    )(feature, w1, b1, adj)


def _k2_body(x1_ref, w2_ref, b2_ref, gatw_ref, a1_ref, a2_ref, adj16_ref,
             wh_ref, f1_ref, f2t_ref, p2_scr, *, heads):
    @pl.when(pl.program_id(0) == 0)
    def _():
        p2_scr[...] = jnp.dot(x1_ref[...], w2_ref[...],
                              preferred_element_type=jnp.float32)

    a = adj16_ref[...].astype(jnp.float32)
    x2 = jnp.dot(a, p2_scr[...], preferred_element_type=jnp.float32)
    x2 = x2 + b2_ref[...]
    for h in range(heads):
        wh = jnp.dot(x2, gatw_ref[h], preferred_element_type=jnp.float32)
        wh_ref[h] = wh
        f1_ref[h] = lax.dot_general(wh, a1_ref[h], (((1,), (1,)), ((), ())),
                                    preferred_element_type=jnp.float32)
        f2t_ref[h] = lax.dot_general(a2_ref[h], wh, (((1,), (1,)), ((), ())),
                                     preferred_element_type=jnp.float32)


def _k2(x1, w2, b2, gat_w, a1, a2, adj16, br):
    n, hid = x1.shape
    heads, feat, ghid = gat_w.shape
    return pl.pallas_call(
        functools.partial(_k2_body, heads=heads),
        grid=(n // br,),
        in_specs=[
            pl.BlockSpec((n, hid), lambda i: (0, 0)),
            pl.BlockSpec((hid, feat), lambda i: (0, 0)),
            pl.BlockSpec((1, feat), lambda i: (0, 0)),
            pl.BlockSpec((heads, feat, ghid), lambda i: (0, 0, 0)),
            pl.BlockSpec((heads, 1, ghid), lambda i: (0, 0, 0)),
            pl.BlockSpec((heads, 1, ghid), lambda i: (0, 0, 0)),
            pl.BlockSpec((br, n), lambda i: (i, 0)),
        ],
        out_specs=[
            pl.BlockSpec((heads, br, ghid), lambda i: (0, i, 0)),
            pl.BlockSpec((heads, br, 1), lambda i: (0, i, 0)),
            pl.BlockSpec((heads, 1, br), lambda i: (0, 0, i)),
        ],
        out_shape=[
            jax.ShapeDtypeStruct((heads, n, ghid), jnp.float32),
            jax.ShapeDtypeStruct((heads, n, 1), jnp.float32),
            jax.ShapeDtypeStruct((heads, 1, n), jnp.float32),
        ],
        scratch_shapes=[pltpu.VMEM((n, feat), jnp.float32)],
    )(x1, w2, b2, gat_w, a1, a2, adj16)


def _k3_body(adj16_ref, wh_ref, f1_ref, f2t_ref, outw_ref, oa1_ref, oa2_ref,
             who_ref, f1o_ref, f2to_ref, *, heads):
    mask = adj16_ref[...] > 0
    cats = []
    for h in range(heads):
        att = _row_softmax_unnorm(f1_ref[h] + f2t_ref[h], mask)
        hp = jnp.dot(att, wh_ref[h], preferred_element_type=jnp.float32)
        cats.append(jnp.where(hp > 0, hp, jnp.exp(hp) - 1.0))
    xcat = jnp.concatenate(cats, axis=1)
    who = jnp.dot(xcat, outw_ref[...], preferred_element_type=jnp.float32)
    who_ref[...] = who
    f1o_ref[...] = lax.dot_general(who, oa1_ref[...], (((1,), (1,)), ((), ())),
                                   preferred_element_type=jnp.float32)
    f2to_ref[...] = lax.dot_general(oa2_ref[...], who, (((1,), (1,)), ((), ())),
                                    preferred_element_type=jnp.float32)


def _k3(adj16, wh_all, f1_all, f2t_all, out_w, oa1, oa2, br):
    heads, n, hid = wh_all.shape
    feat = out_w.shape[1]
    return pl.pallas_call(
        functools.partial(_k3_body, heads=heads),
        grid=(n // br,),
        in_specs=[
            pl.BlockSpec((br, n), lambda i: (i, 0)),
            pl.BlockSpec((heads, n, hid), lambda i: (0, 0, 0)),
            pl.BlockSpec((heads, br, 1), lambda i: (0, i, 0)),
            pl.BlockSpec((heads, 1, n), lambda i: (0, 0, 0)),
            pl.BlockSpec((heads * hid, feat), lambda i: (0, 0)),
            pl.BlockSpec((1, feat), lambda i: (0, 0)),
            pl.BlockSpec((1, feat), lambda i: (0, 0)),
        ],
        out_specs=[
            pl.BlockSpec((br, feat), lambda i: (i, 0)),
            pl.BlockSpec((br, 1), lambda i: (i, 0)),
            pl.BlockSpec((1, br), lambda i: (0, i)),
        ],
        out_shape=[
            jax.ShapeDtypeStruct((n, feat), jnp.float32),
            jax.ShapeDtypeStruct((n, 1), jnp.float32),
            jax.ShapeDtypeStruct((1, n), jnp.float32),
        ],
    )(adj16, wh_all, f1_all, f2t_all, out_w, oa1, oa2)


def _k4_body(adj16_ref, who_ref, f1o_ref, f2to_ref, out_ref):
    mask = adj16_ref[...] > 0
    att = _row_softmax_unnorm(f1o_ref[...] + f2to_ref[...], mask)
    hp = jnp.dot(att, who_ref[...], preferred_element_type=jnp.float32)
    y = jnp.where(hp > 0, hp, jnp.exp(hp) - 1.0)
    my = jnp.max(y, axis=1, keepdims=True)
    sh = y - my
    out_ref[...] = sh - jnp.log(jnp.sum(jnp.exp(sh), axis=1, keepdims=True))


def _k4(adj16, who, f1o, f2to, br):
    n, feat = who.shape
    return pl.pallas_call(
        _k4_body,
        grid=(n // br,),
        in_specs=[
            pl.BlockSpec((br, n), lambda i: (i, 0)),
            pl.BlockSpec((n, feat), lambda i: (0, 0)),
            pl.BlockSpec((br, 1), lambda i: (i, 0)),
            pl.BlockSpec((1, n), lambda i: (0, 0)),
        ],
        out_specs=pl.BlockSpec((br, feat), lambda i: (i, 0)),
        out_shape=jax.ShapeDtypeStruct((n, feat), jnp.float32),
    )(adj16, who, f1o, f2to)


def kernel(feature, adj, gcn_W1, gcn_b1, gcn_W2, gcn_b2, gat_W, gat_a,
           out_W, out_a):
    n, feat = feature.shape
    hid = gcn_W1.shape[1]
    heads = gat_W.shape[0]
    br = 256 if n % 256 == 0 else n

    b1 = gcn_b1.reshape(1, hid)
    b2 = gcn_b2.reshape(1, feat)
    a1 = gat_a[:, :hid, 0].reshape(heads, 1, hid)
    a2 = gat_a[:, hid:, 0].reshape(heads, 1, hid)
    oa1 = out_a[:feat, 0].reshape(1, feat)
    oa2 = out_a[feat:, 0].reshape(1, feat)

    x1, adj16 = _k1(feature, gcn_W1, b1, adj, br)
    wh_all, f1_all, f2t_all = _k2(x1, gcn_W2, b2, gat_W, a1, a2, adj16, br)
    who, f1o, f2to = _k3(adj16, wh_all, f1_all, f2t_all, out_W, oa1, oa2, br)
    x_out = _k4(adj16, who, f1o, f2to, br)

    return (x_out, adj)


# speedup vs baseline: 1.5655x; 1.0322x over previous
import functools

import jax
import jax.numpy as jnp
from jax import lax
from jax.experimental import pallas as pl
from jax.experimental.pallas import tpu as pltpu

ALPHA = 0.1
NEG = -9e15


def _row_softmax_unnorm(e, mask):
    e = jnp.maximum(e, ALPHA * e)
    e = jnp.where(mask, e, NEG)
    m = jnp.max(e, axis=1, keepdims=True)
    ex = jnp.exp(e - m)
    s = jnp.sum(ex, axis=1, keepdims=True)
    return ex * (1.0 / s)


def _k1_body(feat_ref, w1_ref, b1_ref, adj_ref, x1_ref, adj16_ref, p1_scr):
    @pl.when(pl.program_id(0) == 0)
    def _():
        p1_scr[...] = jnp.dot(feat_ref[...], w1_ref[...],
                              preferred_element_type=jnp.float32)

    a = adj_ref[...]
    adj16_ref[...] = a.astype(jnp.bfloat16)
    x = jnp.dot(a, p1_scr[...], preferred_element_type=jnp.float32)
    x1_ref[...] = jnp.maximum(x + b1_ref[...], 0.0)


def _k1(feature, w1, b1, adj, br):
    n, feat = feature.shape
    hid = w1.shape[1]
    return pl.pallas_call(
        _k1_body,
        grid=(n // br,),
        in_specs=[
            pl.BlockSpec((n, feat), lambda i: (0, 0)),
            pl.BlockSpec((feat, hid), lambda i: (0, 0)),
            pl.BlockSpec((1, hid), lambda i: (0, 0)),
            pl.BlockSpec((br, n), lambda i: (i, 0)),
        ],
        out_specs=[
            pl.BlockSpec((br, hid), lambda i: (i, 0)),
            pl.BlockSpec((br, n), lambda i: (i, 0)),
        ],
        out_shape=[
            jax.ShapeDtypeStruct((n, hid), jnp.float32),
            jax.ShapeDtypeStruct((n, n), jnp.bfloat16),
        ],
        scratch_shapes=[pltpu.VMEM((n, hid), jnp.float32)],
    )(feature, w1, b1, adj)


def _k2_body(x1_ref, w2_ref, b2_ref, gatw_ref, a1_ref, a2_ref, adj16_ref,
             wh_ref, f1_ref, f2t_ref, p2_scr, *, heads):
    @pl.when(pl.program_id(0) == 0)
    def _():
        p2_scr[...] = jnp.dot(x1_ref[...], w2_ref[...],
                              preferred_element_type=jnp.float32)

    a = adj16_ref[...].astype(jnp.float32)
    x2 = jnp.dot(a, p2_scr[...], preferred_element_type=jnp.float32)
    x2 = x2 + b2_ref[...]
    for h in range(heads):
        wh = jnp.dot(x2, gatw_ref[h], preferred_element_type=jnp.float32)
        wh_ref[h] = wh
        f1_ref[h] = lax.dot_general(wh, a1_ref[h], (((1,), (1,)), ((), ())),
                                    preferred_element_type=jnp.float32)
        f2t_ref[h] = lax.dot_general(a2_ref[h], wh, (((1,), (1,)), ((), ())),
                                     preferred_element_type=jnp.float32)


def _k2(x1, w2, b2, gat_w, a1, a2, adj16, br):
    n, hid = x1.shape
    heads, feat, ghid = gat_w.shape
    return pl.pallas_call(
        functools.partial(_k2_body, heads=heads),
        grid=(n // br,),
        in_specs=[
            pl.BlockSpec((n, hid), lambda i: (0, 0)),
            pl.BlockSpec((hid, feat), lambda i: (0, 0)),
            pl.BlockSpec((1, feat), lambda i: (0, 0)),
            pl.BlockSpec((heads, feat, ghid), lambda i: (0, 0, 0)),
            pl.BlockSpec((heads, 1, ghid), lambda i: (0, 0, 0)),
            pl.BlockSpec((heads, 1, ghid), lambda i: (0, 0, 0)),
            pl.BlockSpec((br, n), lambda i: (i, 0)),
        ],
        out_specs=[
            pl.BlockSpec((heads, br, ghid), lambda i: (0, i, 0)),
            pl.BlockSpec((heads, br, 1), lambda i: (0, i, 0)),
            pl.BlockSpec((heads, 1, br), lambda i: (0, 0, i)),
        ],
        out_shape=[
            jax.ShapeDtypeStruct((heads, n, ghid), jnp.float32),
            jax.ShapeDtypeStruct((heads, n, 1), jnp.float32),
            jax.ShapeDtypeStruct((heads, 1, n), jnp.float32),
        ],
        scratch_shapes=[pltpu.VMEM((n, feat), jnp.float32)],
    )(x1, w2, b2, gat_w, a1, a2, adj16)


def _k3_body(adj16_ref, wh_ref, f1_ref, f2t_ref, outw_ref, oa1_ref, oa2_ref,
             who_ref, f1o_ref, f2to_ref, *, heads):
    mask = adj16_ref[...] > 0
    cats = []
    for h in range(heads):
        att = _row_softmax_unnorm(f1_ref[h] + f2t_ref[h], mask)
        hp = jnp.dot(att, wh_ref[h], preferred_element_type=jnp.float32)
        cats.append(jnp.where(hp > 0, hp, jnp.exp(hp) - 1.0))
    xcat = jnp.concatenate(cats, axis=1)
    who = jnp.dot(xcat, outw_ref[...], preferred_element_type=jnp.float32)
    who_ref[...] = who
    f1o_ref[...] = lax.dot_general(who, oa1_ref[...], (((1,), (1,)), ((), ())),
                                   preferred_element_type=jnp.float32)
    f2to_ref[...] = lax.dot_general(oa2_ref[...], who, (((1,), (1,)), ((), ())),
                                    preferred_element_type=jnp.float32)


def _k3(adj16, wh_all, f1_all, f2t_all, out_w, oa1, oa2, br):
    heads, n, hid = wh_all.shape
    feat = out_w.shape[1]
    return pl.pallas_call(
        functools.partial(_k3_body, heads=heads),
        grid=(n // br,),
        in_specs=[
            pl.BlockSpec((br, n), lambda i: (i, 0)),
            pl.BlockSpec((heads, n, hid), lambda i: (0, 0, 0)),
            pl.BlockSpec((heads, br, 1), lambda i: (0, i, 0)),
            pl.BlockSpec((heads, 1, n), lambda i: (0, 0, 0)),
            pl.BlockSpec((heads * hid, feat), lambda i: (0, 0)),
            pl.BlockSpec((1, feat), lambda i: (0, 0)),
            pl.BlockSpec((1, feat), lambda i: (0, 0)),
        ],
        out_specs=[
            pl.BlockSpec((br, feat), lambda i: (i, 0)),
            pl.BlockSpec((br, 1), lambda i: (i, 0)),
            pl.BlockSpec((1, br), lambda i: (0, i)),
        ],
        out_shape=[
            jax.ShapeDtypeStruct((n, feat), jnp.float32),
            jax.ShapeDtypeStruct((n, 1), jnp.float32),
            jax.ShapeDtypeStruct((1, n), jnp.float32),
        ],
    )(adj16, wh_all, f1_all, f2t_all, out_w, oa1, oa2)


def _k4_body(adj16_ref, who_ref, f1o_ref, f2to_ref, out_ref):
    mask = adj16_ref[...] > 0
    att = _row_softmax_unnorm(f1o_ref[...] + f2to_ref[...], mask)
    hp = jnp.dot(att, who_ref[...], preferred_element_type=jnp.float32)
    y = jnp.where(hp > 0, hp, jnp.exp(hp) - 1.0)
    my = jnp.max(y, axis=1, keepdims=True)
    sh = y - my
    out_ref[...] = sh - jnp.log(jnp.sum(jnp.exp(sh), axis=1, keepdims=True))


def _k4(adj16, who, f1o, f2to, br):
    n, feat = who.shape
    return pl.pallas_call(
        _k4_body,
        grid=(n // br,),
        in_specs=[
            pl.BlockSpec((br, n), lambda i: (i, 0)),
            pl.BlockSpec((n, feat), lambda i: (0, 0)),
            pl.BlockSpec((br, 1), lambda i: (i, 0)),
            pl.BlockSpec((1, n), lambda i: (0, 0)),
        ],
        out_specs=pl.BlockSpec((br, feat), lambda i: (i, 0)),
        out_shape=jax.ShapeDtypeStruct((n, feat), jnp.float32),
    )(adj16, who, f1o, f2to)


def kernel(feature, adj, gcn_W1, gcn_b1, gcn_W2, gcn_b2, gat_W, gat_a,
           out_W, out_a):
    n, feat = feature.shape
    hid = gcn_W1.shape[1]
    heads = gat_W.shape[0]
    br = 256 if n % 256 == 0 else n
    brw = 512 if n % 512 == 0 else br

    b1 = gcn_b1.reshape(1, hid)
    b2 = gcn_b2.reshape(1, feat)
    a1 = gat_a[:, :hid, 0].reshape(heads, 1, hid)
    a2 = gat_a[:, hid:, 0].reshape(heads, 1, hid)
    oa1 = out_a[:feat, 0].reshape(1, feat)
    oa2 = out_a[feat:, 0].reshape(1, feat)

    x1, adj16 = _k1(feature, gcn_W1, b1, adj, brw)
    wh_all, f1_all, f2t_all = _k2(x1, gcn_W2, b2, gat_W, a1, a2, adj16, brw)
    who, f1o, f2to = _k3(adj16, wh_all, f1_all, f2t_all, out_W, oa1, oa2, br)
    x_out = _k4(adj16, who, f1o, f2to, brw)

    return (x_out, adj)


# speedup vs baseline: 1.5671x; 1.0010x over previous
import functools

import jax
import jax.numpy as jnp
from jax import lax
from jax.experimental import pallas as pl
from jax.experimental.pallas import tpu as pltpu

ALPHA = 0.1
NEG = -9e15


def _row_softmax_unnorm(e, mask):
    e = jnp.maximum(e, ALPHA * e)
    e = jnp.where(mask, e, NEG)
    m = jnp.max(e, axis=1, keepdims=True)
    ex = jnp.exp(e - m)
    s = jnp.sum(ex, axis=1, keepdims=True)
    return ex * (1.0 / s)


def _k1_body(feat_ref, w1_ref, b1_ref, adj_ref, x1_ref, adj16_ref, p1_scr):
    @pl.when(pl.program_id(0) == 0)
    def _():
        p1_scr[...] = jnp.dot(feat_ref[...], w1_ref[...],
                              preferred_element_type=jnp.float32)

    a = adj_ref[...]
    adj16_ref[...] = a.astype(jnp.bfloat16)
    x = jnp.dot(a, p1_scr[...], preferred_element_type=jnp.float32)
    x1_ref[...] = jnp.maximum(x + b1_ref[...], 0.0)


def _k1(feature, w1, b1, adj, br):
    n, feat = feature.shape
    hid = w1.shape[1]
    return pl.pallas_call(
        _k1_body,
        grid=(n // br,),
        in_specs=[
            pl.BlockSpec((n, feat), lambda i: (0, 0)),
            pl.BlockSpec((feat, hid), lambda i: (0, 0)),
            pl.BlockSpec((1, hid), lambda i: (0, 0)),
            pl.BlockSpec((br, n), lambda i: (i, 0)),
        ],
        out_specs=[
            pl.BlockSpec((br, hid), lambda i: (i, 0)),
            pl.BlockSpec((br, n), lambda i: (i, 0)),
        ],
        out_shape=[
            jax.ShapeDtypeStruct((n, hid), jnp.float32),
            jax.ShapeDtypeStruct((n, n), jnp.bfloat16),
        ],
        scratch_shapes=[pltpu.VMEM((n, hid), jnp.float32)],
    )(feature, w1, b1, adj)


def _k2_body(x1_ref, w2_ref, b2_ref, gatw_ref, a1_ref, a2_ref, adj16_ref,
             wh_ref, f1_ref, f2t_ref, p2_scr, *, heads):
    @pl.when(pl.program_id(0) == 0)
    def _():
        p2_scr[...] = jnp.dot(x1_ref[...], w2_ref[...],
                              preferred_element_type=jnp.float32)

    a = adj16_ref[...].astype(jnp.float32)
    x2 = jnp.dot(a, p2_scr[...], preferred_element_type=jnp.float32)
    x2 = x2 + b2_ref[...]
    for h in range(heads):
        wh = jnp.dot(x2, gatw_ref[h], preferred_element_type=jnp.float32)
        wh_ref[h] = wh.astype(jnp.bfloat16)
        f1_ref[h] = lax.dot_general(wh, a1_ref[h], (((1,), (1,)), ((), ())),
                                    preferred_element_type=jnp.float32)
        f2t_ref[h] = lax.dot_general(a2_ref[h], wh, (((1,), (1,)), ((), ())),
                                     preferred_element_type=jnp.float32)


def _k2(x1, w2, b2, gat_w, a1, a2, adj16, br):
    n, hid = x1.shape
    heads, feat, ghid = gat_w.shape
    return pl.pallas_call(
        functools.partial(_k2_body, heads=heads),
        grid=(n // br,),
        in_specs=[
            pl.BlockSpec((n, hid), lambda i: (0, 0)),
            pl.BlockSpec((hid, feat), lambda i: (0, 0)),
            pl.BlockSpec((1, feat), lambda i: (0, 0)),
            pl.BlockSpec((heads, feat, ghid), lambda i: (0, 0, 0)),
            pl.BlockSpec((heads, 1, ghid), lambda i: (0, 0, 0)),
            pl.BlockSpec((heads, 1, ghid), lambda i: (0, 0, 0)),
            pl.BlockSpec((br, n), lambda i: (i, 0)),
        ],
        out_specs=[
            pl.BlockSpec((heads, br, ghid), lambda i: (0, i, 0)),
            pl.BlockSpec((heads, br, 1), lambda i: (0, i, 0)),
            pl.BlockSpec((heads, 1, br), lambda i: (0, 0, i)),
        ],
        out_shape=[
            jax.ShapeDtypeStruct((heads, n, ghid), jnp.bfloat16),
            jax.ShapeDtypeStruct((heads, n, 1), jnp.float32),
            jax.ShapeDtypeStruct((heads, 1, n), jnp.float32),
        ],
        scratch_shapes=[pltpu.VMEM((n, feat), jnp.float32)],
    )(x1, w2, b2, gat_w, a1, a2, adj16)


def _k3_body(adj16_ref, wh_ref, f1_ref, f2t_ref, outw_ref, oa1_ref, oa2_ref,
             who_ref, f1o_ref, f2to_ref, *, heads):
    mask = adj16_ref[...] > 0
    cats = []
    for h in range(heads):
        att = _row_softmax_unnorm(f1_ref[h] + f2t_ref[h], mask)
        hp = jnp.dot(att, wh_ref[h], preferred_element_type=jnp.float32)
        cats.append(jnp.where(hp > 0, hp, jnp.exp(hp) - 1.0))
    xcat = jnp.concatenate(cats, axis=1)
    who = jnp.dot(xcat, outw_ref[...], preferred_element_type=jnp.float32)
    who_ref[...] = who
    f1o_ref[...] = lax.dot_general(who, oa1_ref[...], (((1,), (1,)), ((), ())),
                                   preferred_element_type=jnp.float32)
    f2to_ref[...] = lax.dot_general(oa2_ref[...], who, (((1,), (1,)), ((), ())),
                                    preferred_element_type=jnp.float32)


def _k3(adj16, wh_all, f1_all, f2t_all, out_w, oa1, oa2, br):
    heads, n, hid = wh_all.shape
    feat = out_w.shape[1]
    return pl.pallas_call(
        functools.partial(_k3_body, heads=heads),
        grid=(n // br,),
        in_specs=[
            pl.BlockSpec((br, n), lambda i: (i, 0)),
            pl.BlockSpec((heads, n, hid), lambda i: (0, 0, 0)),
            pl.BlockSpec((heads, br, 1), lambda i: (0, i, 0)),
            pl.BlockSpec((heads, 1, n), lambda i: (0, 0, 0)),
            pl.BlockSpec((heads * hid, feat), lambda i: (0, 0)),
            pl.BlockSpec((1, feat), lambda i: (0, 0)),
            pl.BlockSpec((1, feat), lambda i: (0, 0)),
        ],
        out_specs=[
            pl.BlockSpec((br, feat), lambda i: (i, 0)),
            pl.BlockSpec((br, 1), lambda i: (i, 0)),
            pl.BlockSpec((1, br), lambda i: (0, i)),
        ],
        out_shape=[
            jax.ShapeDtypeStruct((n, feat), jnp.float32),
            jax.ShapeDtypeStruct((n, 1), jnp.float32),
            jax.ShapeDtypeStruct((1, n), jnp.float32),
        ],
    )(adj16, wh_all, f1_all, f2t_all, out_w, oa1, oa2)


def _k4_body(adj16_ref, who_ref, f1o_ref, f2to_ref, out_ref):
    mask = adj16_ref[...] > 0
    att = _row_softmax_unnorm(f1o_ref[...] + f2to_ref[...], mask)
    hp = jnp.dot(att, who_ref[...], preferred_element_type=jnp.float32)
    y = jnp.where(hp > 0, hp, jnp.exp(hp) - 1.0)
    my = jnp.max(y, axis=1, keepdims=True)
    sh = y - my
    out_ref[...] = sh - jnp.log(jnp.sum(jnp.exp(sh), axis=1, keepdims=True))


def _k4(adj16, who, f1o, f2to, br):
    n, feat = who.shape
    return pl.pallas_call(
        _k4_body,
        grid=(n // br,),
        in_specs=[
            pl.BlockSpec((br, n), lambda i: (i, 0)),
            pl.BlockSpec((n, feat), lambda i: (0, 0)),
            pl.BlockSpec((br, 1), lambda i: (i, 0)),
            pl.BlockSpec((1, n), lambda i: (0, 0)),
        ],
        out_specs=pl.BlockSpec((br, feat), lambda i: (i, 0)),
        out_shape=jax.ShapeDtypeStruct((n, feat), jnp.float32),
    )(adj16, who, f1o, f2to)


def kernel(feature, adj, gcn_W1, gcn_b1, gcn_W2, gcn_b2, gat_W, gat_a,
           out_W, out_a):
    n, feat = feature.shape
    hid = gcn_W1.shape[1]
    heads = gat_W.shape[0]
    br = 256 if n % 256 == 0 else n
    brw = 512 if n % 512 == 0 else br

    b1 = gcn_b1.reshape(1, hid)
    b2 = gcn_b2.reshape(1, feat)
    a1 = gat_a[:, :hid, 0].reshape(heads, 1, hid)
    a2 = gat_a[:, hid:, 0].reshape(heads, 1, hid)
    oa1 = out_a[:feat, 0].reshape(1, feat)
    oa2 = out_a[feat:, 0].reshape(1, feat)

    x1, adj16 = _k1(feature, gcn_W1, b1, adj, brw)
    wh_all, f1_all, f2t_all = _k2(x1, gcn_W2, b2, gat_W, a1, a2, adj16, brw)
    who, f1o, f2to = _k3(adj16, wh_all, f1_all, f2t_all, out_W, oa1, oa2, br)
    x_out = _k4(adj16, who, f1o, f2to, brw)

    return (x_out, adj)
